# baseline (speedup 1.0000x reference)
import functools

import jax
import jax.numpy as jnp
from jax import lax
from jax.experimental import pallas as pl
from jax.experimental.pallas import tpu as pltpu

VOCAB = 1000
NEMBED = 32
B, T = 1024, 50
BB = 8
NB = B // BB


def _tc_body(idx_ref, tok_ref, pos_ref, w_ref, b_ref, out_ref):
    tok = tok_ref[...]
    pos = pos_ref[...]
    w = w_ref[...]
    bias = b_ref[...]
    for r in range(BB):
        ids = idx_ref[0, r, :]
        onehot = (ids[:, None] ==
                  lax.broadcasted_iota(jnp.int32, (T, VOCAB), 1)
                  ).astype(jnp.float32)
        x = jnp.dot(onehot, tok, preferred_element_type=jnp.float32)
        x = x + pos
        y = jnp.dot(x, w, preferred_element_type=jnp.float32) + bias
        out_ref[r, :, :] = y


def kernel(idx, token_table, pos_table, W, b):
    idx3 = idx.reshape(NB, BB, T)
    pos = pos_table[:T]
    b2 = b.reshape(1, VOCAB)
    out = pl.pallas_call(
        _tc_body,
        grid=(NB,),
        in_specs=[
            pl.BlockSpec((1, BB, T), lambda i: (i, 0, 0)),
            pl.BlockSpec((VOCAB, NEMBED), lambda i: (0, 0)),
            pl.BlockSpec((T, NEMBED), lambda i: (0, 0)),
            pl.BlockSpec((NEMBED, VOCAB), lambda i: (0, 0)),
            pl.BlockSpec((1, VOCAB), lambda i: (0, 0)),
        ],
        out_specs=pl.BlockSpec((BB, T, VOCAB), lambda i: (i, 0, 0)),
        out_shape=jax.ShapeDtypeStruct((B, T, VOCAB), jnp.float32),
    )(idx3, token_table, pos, W, b2)
    return out
